# baseline (speedup 1.0000x reference)
import functools

import jax
import jax.numpy as jnp
from jax import lax
from jax.experimental import pallas as pl
from jax.experimental.pallas import tpu as pltpu
from jax.experimental.pallas import tpu_sc as plsc

N_NODES = 10000
N_EDGES = 160000
DIM = 256
NODE_VOCAB = 5000
EDGE_VOCAB = 500
NODE_VOCAB_PAD = 5120
EDGE_VOCAB_PAD = 512
EPS = 1e-5

NC = 2
NS = 16
NW = NC * NS
L = 16

_MESH = plsc.VectorSubcoreMesh(core_axis_name="c", subcore_axis_name="s")

_NODE_CHUNK = 320
_NODE_TAIL = N_NODES - 31 * _NODE_CHUNK
_EDGE_CHUNK = N_EDGES // NW
_EDGE_FULL = _EDGE_CHUNK // L
_EDGE_REM = _EDGE_CHUNK - _EDGE_FULL * L


@functools.partial(
    pl.kernel,
    out_type=(
        jax.ShapeDtypeStruct((NW, NODE_VOCAB_PAD), jnp.float32),
        jax.ShapeDtypeStruct((NW, EDGE_VOCAB_PAD), jnp.float32),
    ),
    mesh=_MESH,
    scratch_types=[
        pltpu.VMEM((_NODE_CHUNK,), jnp.int32),
        pltpu.VMEM((_EDGE_CHUNK + 16,), jnp.int32),
        pltpu.VMEM((NS * NODE_VOCAB_PAD,), jnp.float32),
        pltpu.VMEM((NS * EDGE_VOCAB_PAD,), jnp.float32),
        pltpu.VMEM((NODE_VOCAB_PAD,), jnp.float32),
        pltpu.VMEM((EDGE_VOCAB_PAD,), jnp.float32),
    ],
    compiler_params=pltpu.CompilerParams(needs_layout_passes=False),
)
def _hist(x_h, e_h, cn_out, ce_out, idxn_v, idxe_v, cntn_v, cnte_v, redn_v,
          rede_v):
    wid = lax.axis_index("s") * NC + lax.axis_index("c")
    lanes = jnp.arange(L, dtype=jnp.int32)
    ones = jnp.ones((L,), jnp.float32)
    zeros = jnp.zeros((L,), jnp.float32)
    lane_off_n = lanes * NODE_VOCAB_PAD
    lane_off_e = lanes * EDGE_VOCAB_PAD

    def zn(c, carry):
        cntn_v[pl.ds(c * L, L)] = zeros
        return carry
    lax.fori_loop(0, NS * NODE_VOCAB_PAD // L, zn, 0)
    def ze(c, carry):
        cnte_v[pl.ds(c * L, L)] = zeros
        return carry
    lax.fori_loop(0, NS * EDGE_VOCAB_PAD // L, ze, 0)

    @pl.when(wid < 31)
    def _():
        pltpu.sync_copy(x_h.at[pl.ds(wid * _NODE_CHUNK, _NODE_CHUNK)], idxn_v)
        def sn(i, carry):
            iv = idxn_v[pl.ds(i * L, L)]
            plsc.addupdate_scatter(cntn_v, [lane_off_n + iv], ones)
            return carry
        lax.fori_loop(0, _NODE_CHUNK // L, sn, 0)

    @pl.when(wid == 31)
    def _():
        pltpu.sync_copy(x_h.at[pl.ds(31 * _NODE_CHUNK, _NODE_TAIL)],
                        idxn_v.at[pl.ds(0, _NODE_TAIL)])
        def sn(i, carry):
            iv = idxn_v[pl.ds(i * L, L)]
            plsc.addupdate_scatter(cntn_v, [lane_off_n + iv], ones)
            return carry
        lax.fori_loop(0, _NODE_TAIL // L, sn, 0)

    pltpu.sync_copy(e_h.at[pl.ds(wid * _EDGE_CHUNK, _EDGE_CHUNK)],
                    idxe_v.at[pl.ds(0, _EDGE_CHUNK)])
    def se(i, carry):
        iv = idxe_v[pl.ds(i * L, L)]
        plsc.addupdate_scatter(cnte_v, [lane_off_e + iv], ones)
        return carry
    lax.fori_loop(0, _EDGE_FULL, se, 0)
    iv = idxe_v[pl.ds(_EDGE_FULL * L, L)]
    plsc.addupdate_scatter(cnte_v, [lane_off_e + iv], ones,
                           mask=lanes < _EDGE_REM)

    def rn(c, carry):
        s = pl.ds(c * L, L)
        acc = cntn_v[s]
        for r in range(1, NS):
            acc = acc + cntn_v[pl.ds(r * NODE_VOCAB_PAD + c * L, L)]
        redn_v[s] = acc
        return carry
    lax.fori_loop(0, NODE_VOCAB_PAD // L, rn, 0)
    for c in range(EDGE_VOCAB_PAD // L):
        s = pl.ds(c * L, L)
        acc = cnte_v[s]
        for r in range(1, NS):
            acc = acc + cnte_v[pl.ds(r * EDGE_VOCAB_PAD + c * L, L)]
        rede_v[s] = acc

    pltpu.sync_copy(redn_v, cn_out.at[wid])
    pltpu.sync_copy(rede_v, ce_out.at[wid])


def _stats_body(cn_ref, ce_ref, nt_ref, et_ref, ntn_ref, etn_ref):
    for c_ref, t_ref, o_ref, n in (
        (cn_ref, nt_ref, ntn_ref, float(N_NODES)),
        (ce_ref, et_ref, etn_ref, float(N_EDGES)),
    ):
        counts = jnp.sum(c_ref[...], axis=0, keepdims=True)
        t = t_ref[...]
        mu = lax.dot(counts, t, precision=lax.Precision.HIGHEST) / n
        m2 = lax.dot(counts, t * t, precision=lax.Precision.HIGHEST) / n
        scale = lax.rsqrt(m2 - mu * mu + EPS)
        o_ref[...] = (t - mu) * scale


def _stats(cn, ce, nt_pad, et_pad):
    return pl.pallas_call(
        _stats_body,
        out_shape=(
            jax.ShapeDtypeStruct((NODE_VOCAB_PAD, DIM), jnp.float32),
            jax.ShapeDtypeStruct((EDGE_VOCAB_PAD, DIM), jnp.float32),
        ),
    )(cn, ce, nt_pad, et_pad)


_BLK = 200
_NBLK = (N_NODES + N_EDGES) // _BLK
_NODE_BLKS = N_NODES // _BLK
_ITERS = -(-_NBLK // NW)
_NT_SLICE = NODE_VOCAB_PAD // NS
_ET_SLICE = EDGE_VOCAB_PAD // NS


@functools.partial(
    pl.kernel,
    out_type=jax.ShapeDtypeStruct((N_NODES + N_EDGES, DIM), jnp.float32),
    mesh=_MESH,
    scratch_types=[
        pltpu.VMEM((_BLK,), jnp.int32),
        pltpu.VMEM((_BLK,), jnp.int32),
        pltpu.VMEM((_BLK, DIM), jnp.float32),
        pltpu.VMEM((_BLK, DIM), jnp.float32),
        pltpu.SemaphoreType.DMA,
        pltpu.SemaphoreType.DMA,
        pltpu.SemaphoreType.DMA,
        pltpu.SemaphoreType.DMA,
    ],
)
def _gather(x_h, e_h, nt_h, et_h, out_h, idx0_v, idx1_v, rows0_v,
            rows1_v, g0_sem, g1_sem, w0_sem, w1_sem):
    wid = lax.axis_index("s") * NC + lax.axis_index("c")
    idx_v = (idx0_v, idx1_v)
    rows_v = (rows0_v, rows1_v)
    gsem = (g0_sem, g1_sem)
    wsem = (w0_sem, w1_sem)

    def start_block(i):
        buf = i & 1
        b = wid + NW * i

        @pl.when(b < _NBLK)
        def _():
            if i >= 2:
                pltpu.make_async_copy(
                    rows_v[buf], out_h.at[pl.ds(0, _BLK)], wsem[buf]).wait()

            @pl.when(b < _NODE_BLKS)
            def _():
                pltpu.sync_copy(x_h.at[pl.ds(b * _BLK, _BLK)],
                                idx_v[buf])
                pltpu.async_copy(nt_h.at[idx_v[buf]], rows_v[buf],
                                 gsem[buf])

            @pl.when(b >= _NODE_BLKS)
            def _():
                pltpu.sync_copy(e_h.at[pl.ds(b * _BLK - N_NODES, _BLK)],
                                idx_v[buf])
                pltpu.async_copy(et_h.at[idx_v[buf]], rows_v[buf],
                                 gsem[buf])

    def finish_block(i):
        buf = i & 1
        b = wid + NW * i

        @pl.when(b < _NBLK)
        def _():
            pltpu.make_async_copy(
                nt_h.at[pl.ds(0, _BLK)], rows_v[buf], gsem[buf]).wait()
            pltpu.async_copy(rows_v[buf], out_h.at[pl.ds(b * _BLK, _BLK)],
                             wsem[buf])

    start_block(0)
    for i in range(1, _ITERS):
        start_block(i)
        finish_block(i - 1)
    finish_block(_ITERS - 1)
    for i in (_ITERS - 2, _ITERS - 1):
        buf = i & 1
        b = wid + NW * i

        @pl.when(b < _NBLK)
        def _():
            pltpu.make_async_copy(
                rows_v[buf], out_h.at[pl.ds(0, _BLK)], wsem[buf]).wait()


def kernel(x, edge_attr, node_table, edge_table):
    cn, ce = _hist(x, edge_attr)
    nt_pad = jnp.pad(node_table, ((0, NODE_VOCAB_PAD - NODE_VOCAB), (0, 0)))
    et_pad = jnp.pad(edge_table, ((0, EDGE_VOCAB_PAD - EDGE_VOCAB), (0, 0)))
    ntn, etn = _stats(cn, ce, nt_pad, et_pad)
    return _gather(x, edge_attr, ntn, etn)


# speedup vs baseline: 1.0822x; 1.0822x over previous
import functools

import jax
import jax.numpy as jnp
from jax import lax
from jax.experimental import pallas as pl
from jax.experimental.pallas import tpu as pltpu
from jax.experimental.pallas import tpu_sc as plsc

N_NODES = 10000
N_EDGES = 160000
DIM = 256
NODE_VOCAB = 5000
EDGE_VOCAB = 500
NODE_VOCAB_PAD = 5120
EDGE_VOCAB_PAD = 512
EPS = 1e-5

NC = 2
NS = 16
NW = NC * NS
L = 16

_MESH = plsc.VectorSubcoreMesh(core_axis_name="c", subcore_axis_name="s")

_NODE_CHUNK = 320
_NODE_TAIL = N_NODES - 31 * _NODE_CHUNK
_EDGE_CHUNK = N_EDGES // NW
_EDGE_FULL = _EDGE_CHUNK // L
_EDGE_REM = _EDGE_CHUNK - _EDGE_FULL * L


@functools.partial(
    pl.kernel,
    out_type=(
        jax.ShapeDtypeStruct((NW, NODE_VOCAB_PAD), jnp.float32),
        jax.ShapeDtypeStruct((NW, EDGE_VOCAB_PAD), jnp.float32),
    ),
    mesh=_MESH,
    scratch_types=[
        pltpu.VMEM((_NODE_CHUNK,), jnp.int32),
        pltpu.VMEM((_EDGE_CHUNK + 16,), jnp.int32),
        pltpu.VMEM((NS * NODE_VOCAB_PAD,), jnp.float32),
        pltpu.VMEM((NS * EDGE_VOCAB_PAD,), jnp.float32),
        pltpu.VMEM((NODE_VOCAB_PAD,), jnp.float32),
        pltpu.VMEM((EDGE_VOCAB_PAD,), jnp.float32),
    ],
    compiler_params=pltpu.CompilerParams(needs_layout_passes=False),
)
def _hist(x_h, e_h, cn_out, ce_out, idxn_v, idxe_v, cntn_v, cnte_v, redn_v,
          rede_v):
    wid = lax.axis_index("s") * NC + lax.axis_index("c")
    lanes = jnp.arange(L, dtype=jnp.int32)
    ones = jnp.ones((L,), jnp.float32)
    zeros = jnp.zeros((L,), jnp.float32)
    lane_off_n = lanes * NODE_VOCAB_PAD
    lane_off_e = lanes * EDGE_VOCAB_PAD

    def zn(c, carry):
        cntn_v[pl.ds(c * L, L)] = zeros
        return carry
    lax.fori_loop(0, NS * NODE_VOCAB_PAD // L, zn, 0, unroll=8)
    def ze(c, carry):
        cnte_v[pl.ds(c * L, L)] = zeros
        return carry
    lax.fori_loop(0, NS * EDGE_VOCAB_PAD // L, ze, 0, unroll=8)

    @pl.when(wid < 31)
    def _():
        pltpu.sync_copy(x_h.at[pl.ds(wid * _NODE_CHUNK, _NODE_CHUNK)], idxn_v)
        def sn(i, carry):
            iv = idxn_v[pl.ds(i * L, L)]
            plsc.addupdate_scatter(cntn_v, [lane_off_n + iv], ones)
            return carry
        lax.fori_loop(0, _NODE_CHUNK // L, sn, 0)

    @pl.when(wid == 31)
    def _():
        pltpu.sync_copy(x_h.at[pl.ds(31 * _NODE_CHUNK, _NODE_TAIL)],
                        idxn_v.at[pl.ds(0, _NODE_TAIL)])
        def sn(i, carry):
            iv = idxn_v[pl.ds(i * L, L)]
            plsc.addupdate_scatter(cntn_v, [lane_off_n + iv], ones)
            return carry
        lax.fori_loop(0, _NODE_TAIL // L, sn, 0)

    pltpu.sync_copy(e_h.at[pl.ds(wid * _EDGE_CHUNK, _EDGE_CHUNK)],
                    idxe_v.at[pl.ds(0, _EDGE_CHUNK)])
    def se(i, carry):
        iv = idxe_v[pl.ds(i * L, L)]
        plsc.addupdate_scatter(cnte_v, [lane_off_e + iv], ones)
        return carry
    lax.fori_loop(0, _EDGE_FULL, se, 0, unroll=8)
    iv = idxe_v[pl.ds(_EDGE_FULL * L, L)]
    plsc.addupdate_scatter(cnte_v, [lane_off_e + iv], ones,
                           mask=lanes < _EDGE_REM)

    def rn(c, carry):
        s = pl.ds(c * L, L)
        acc = cntn_v[s]
        for r in range(1, NS):
            acc = acc + cntn_v[pl.ds(r * NODE_VOCAB_PAD + c * L, L)]
        redn_v[s] = acc
        return carry
    lax.fori_loop(0, NODE_VOCAB_PAD // L, rn, 0, unroll=4)
    for c in range(EDGE_VOCAB_PAD // L):
        s = pl.ds(c * L, L)
        acc = cnte_v[s]
        for r in range(1, NS):
            acc = acc + cnte_v[pl.ds(r * EDGE_VOCAB_PAD + c * L, L)]
        rede_v[s] = acc

    pltpu.sync_copy(redn_v, cn_out.at[wid])
    pltpu.sync_copy(rede_v, ce_out.at[wid])


def _stats_body(cn_ref, ce_ref, nt_ref, et_ref, ntn_ref, etn_ref):
    for c_ref, t_ref, o_ref, n in (
        (cn_ref, nt_ref, ntn_ref, float(N_NODES)),
        (ce_ref, et_ref, etn_ref, float(N_EDGES)),
    ):
        counts = jnp.sum(c_ref[...], axis=0, keepdims=True)
        t = t_ref[...]
        mu = lax.dot(counts, t, precision=lax.Precision.HIGHEST) / n
        m2 = lax.dot(counts, t * t, precision=lax.Precision.HIGHEST) / n
        scale = lax.rsqrt(m2 - mu * mu + EPS)
        o_ref[...] = (t - mu) * scale


def _stats(cn, ce, nt_pad, et_pad):
    return pl.pallas_call(
        _stats_body,
        out_shape=(
            jax.ShapeDtypeStruct((NODE_VOCAB_PAD, DIM), jnp.float32),
            jax.ShapeDtypeStruct((EDGE_VOCAB_PAD, DIM), jnp.float32),
        ),
    )(cn, ce, nt_pad, et_pad)


_BLK = 200
_NBLK = (N_NODES + N_EDGES) // _BLK
_NODE_BLKS = N_NODES // _BLK
_ITERS = -(-_NBLK // NW)
_NT_SLICE = NODE_VOCAB_PAD // NS
_ET_SLICE = EDGE_VOCAB_PAD // NS


@functools.partial(
    pl.kernel,
    out_type=jax.ShapeDtypeStruct((N_NODES + N_EDGES, DIM), jnp.float32),
    mesh=_MESH,
    scratch_types=[
        pltpu.VMEM((_BLK,), jnp.int32),
        pltpu.VMEM((_BLK,), jnp.int32),
        pltpu.VMEM((_BLK, DIM), jnp.float32),
        pltpu.VMEM((_BLK, DIM), jnp.float32),
        pltpu.SemaphoreType.DMA,
        pltpu.SemaphoreType.DMA,
        pltpu.SemaphoreType.DMA,
        pltpu.SemaphoreType.DMA,
    ],
)
def _gather(x_h, e_h, nt_h, et_h, out_h, idx0_v, idx1_v, rows0_v,
            rows1_v, g0_sem, g1_sem, w0_sem, w1_sem):
    wid = lax.axis_index("s") * NC + lax.axis_index("c")
    idx_v = (idx0_v, idx1_v)
    rows_v = (rows0_v, rows1_v)
    gsem = (g0_sem, g1_sem)
    wsem = (w0_sem, w1_sem)

    def start_block(i):
        buf = i & 1
        b = wid + NW * i

        @pl.when(b < _NBLK)
        def _():
            if i >= 2:
                pltpu.make_async_copy(
                    rows_v[buf], out_h.at[pl.ds(0, _BLK)], wsem[buf]).wait()

            @pl.when(b < _NODE_BLKS)
            def _():
                pltpu.sync_copy(x_h.at[pl.ds(b * _BLK, _BLK)],
                                idx_v[buf])
                pltpu.async_copy(nt_h.at[idx_v[buf]], rows_v[buf],
                                 gsem[buf])

            @pl.when(b >= _NODE_BLKS)
            def _():
                pltpu.sync_copy(e_h.at[pl.ds(b * _BLK - N_NODES, _BLK)],
                                idx_v[buf])
                pltpu.async_copy(et_h.at[idx_v[buf]], rows_v[buf],
                                 gsem[buf])

    def finish_block(i):
        buf = i & 1
        b = wid + NW * i

        @pl.when(b < _NBLK)
        def _():
            pltpu.make_async_copy(
                nt_h.at[pl.ds(0, _BLK)], rows_v[buf], gsem[buf]).wait()
            pltpu.async_copy(rows_v[buf], out_h.at[pl.ds(b * _BLK, _BLK)],
                             wsem[buf])

    start_block(0)
    for i in range(1, _ITERS):
        start_block(i)
        finish_block(i - 1)
    finish_block(_ITERS - 1)
    for i in (_ITERS - 2, _ITERS - 1):
        buf = i & 1
        b = wid + NW * i

        @pl.when(b < _NBLK)
        def _():
            pltpu.make_async_copy(
                rows_v[buf], out_h.at[pl.ds(0, _BLK)], wsem[buf]).wait()


def kernel(x, edge_attr, node_table, edge_table):
    cn, ce = _hist(x, edge_attr)
    nt_pad = jnp.pad(node_table, ((0, NODE_VOCAB_PAD - NODE_VOCAB), (0, 0)))
    et_pad = jnp.pad(edge_table, ((0, EDGE_VOCAB_PAD - EDGE_VOCAB), (0, 0)))
    ntn, etn = _stats(cn, ce, nt_pad, et_pad)
    return _gather(x, edge_attr, ntn, etn)


# speedup vs baseline: 1.2457x; 1.1512x over previous
import functools

import jax
import jax.numpy as jnp
from jax import lax
from jax.experimental import pallas as pl
from jax.experimental.pallas import tpu as pltpu
from jax.experimental.pallas import tpu_sc as plsc

N_NODES = 10000
N_EDGES = 160000
DIM = 256
NODE_VOCAB = 5000
EDGE_VOCAB = 500
NODE_VOCAB_PAD = 5120
EDGE_VOCAB_PAD = 512
EPS = 1e-5

NC = 2
NS = 16
NW = NC * NS
L = 16

_MESH = plsc.VectorSubcoreMesh(core_axis_name="c", subcore_axis_name="s")

_NODE_CHUNK = 320
_NODE_TAIL = N_NODES - 31 * _NODE_CHUNK
_EDGE_CHUNK = N_EDGES // NW
_EDGE_FULL = _EDGE_CHUNK // L
_EDGE_REM = _EDGE_CHUNK - _EDGE_FULL * L


@functools.partial(
    pl.kernel,
    out_type=(
        jax.ShapeDtypeStruct((NW, NODE_VOCAB_PAD), jnp.float32),
        jax.ShapeDtypeStruct((NW, EDGE_VOCAB_PAD), jnp.float32),
    ),
    mesh=_MESH,
    scratch_types=[
        pltpu.VMEM((_NODE_CHUNK,), jnp.int32),
        pltpu.VMEM((_EDGE_CHUNK + 16,), jnp.int32),
        pltpu.VMEM((NS * NODE_VOCAB_PAD,), jnp.float32),
        pltpu.VMEM((NS * EDGE_VOCAB_PAD,), jnp.float32),
        pltpu.VMEM((NODE_VOCAB_PAD,), jnp.float32),
        pltpu.VMEM((EDGE_VOCAB_PAD,), jnp.float32),
    ],
    compiler_params=pltpu.CompilerParams(needs_layout_passes=False),
)
def _hist(x_h, e_h, cn_out, ce_out, idxn_v, idxe_v, cntn_v, cnte_v, redn_v,
          rede_v):
    wid = lax.axis_index("s") * NC + lax.axis_index("c")
    lanes = jnp.arange(L, dtype=jnp.int32)
    ones = jnp.ones((L,), jnp.float32)
    zeros = jnp.zeros((L,), jnp.float32)
    lane_off_n = lanes * NODE_VOCAB_PAD
    lane_off_e = lanes * EDGE_VOCAB_PAD

    def zn(c, carry):
        cntn_v[pl.ds(c * L, L)] = zeros
        return carry
    lax.fori_loop(0, NS * NODE_VOCAB_PAD // L, zn, 0, unroll=8)
    def ze(c, carry):
        cnte_v[pl.ds(c * L, L)] = zeros
        return carry
    lax.fori_loop(0, NS * EDGE_VOCAB_PAD // L, ze, 0, unroll=8)

    @pl.when(wid < 31)
    def _():
        pltpu.sync_copy(x_h.at[pl.ds(wid * _NODE_CHUNK, _NODE_CHUNK)], idxn_v)
        def sn(i, carry):
            iv = idxn_v[pl.ds(i * L, L)]
            plsc.addupdate_scatter(cntn_v, [lane_off_n + iv], ones)
            return carry
        lax.fori_loop(0, _NODE_CHUNK // L, sn, 0)

    @pl.when(wid == 31)
    def _():
        pltpu.sync_copy(x_h.at[pl.ds(31 * _NODE_CHUNK, _NODE_TAIL)],
                        idxn_v.at[pl.ds(0, _NODE_TAIL)])
        def sn(i, carry):
            iv = idxn_v[pl.ds(i * L, L)]
            plsc.addupdate_scatter(cntn_v, [lane_off_n + iv], ones)
            return carry
        lax.fori_loop(0, _NODE_TAIL // L, sn, 0)

    pltpu.sync_copy(e_h.at[pl.ds(wid * _EDGE_CHUNK, _EDGE_CHUNK)],
                    idxe_v.at[pl.ds(0, _EDGE_CHUNK)])
    def se(i, carry):
        iv = idxe_v[pl.ds(i * L, L)]
        plsc.addupdate_scatter(cnte_v, [lane_off_e + iv], ones)
        return carry
    lax.fori_loop(0, _EDGE_FULL, se, 0, unroll=8)
    iv = idxe_v[pl.ds(_EDGE_FULL * L, L)]
    plsc.addupdate_scatter(cnte_v, [lane_off_e + iv], ones,
                           mask=lanes < _EDGE_REM)

    def rn(c, carry):
        s = pl.ds(c * L, L)
        acc = cntn_v[s]
        for r in range(1, NS):
            acc = acc + cntn_v[pl.ds(r * NODE_VOCAB_PAD + c * L, L)]
        redn_v[s] = acc
        return carry
    lax.fori_loop(0, NODE_VOCAB_PAD // L, rn, 0, unroll=4)
    for c in range(EDGE_VOCAB_PAD // L):
        s = pl.ds(c * L, L)
        acc = cnte_v[s]
        for r in range(1, NS):
            acc = acc + cnte_v[pl.ds(r * EDGE_VOCAB_PAD + c * L, L)]
        rede_v[s] = acc

    pltpu.sync_copy(redn_v, cn_out.at[wid])
    pltpu.sync_copy(rede_v, ce_out.at[wid])


def _stats_body(cn_ref, ce_ref, nt_ref, et_ref, ntn_ref, etn_ref):
    for c_ref, t_ref, o_ref, n in (
        (cn_ref, nt_ref, ntn_ref, float(N_NODES)),
        (ce_ref, et_ref, etn_ref, float(N_EDGES)),
    ):
        counts = jnp.sum(c_ref[...], axis=0, keepdims=True)[:, :t_ref.shape[0]]
        t = t_ref[...]
        mu = lax.dot(counts, t, precision=lax.Precision.HIGHEST) / n
        m2 = lax.dot(counts, t * t, precision=lax.Precision.HIGHEST) / n
        scale = lax.rsqrt(m2 - mu * mu + EPS)
        o_ref[...] = (t - mu) * scale


def _stats(cn, ce, nt, et):
    return pl.pallas_call(
        _stats_body,
        out_shape=(
            jax.ShapeDtypeStruct((NODE_VOCAB, DIM), jnp.float32),
            jax.ShapeDtypeStruct((EDGE_VOCAB, DIM), jnp.float32),
        ),
    )(cn, ce, nt, et)


_BLK = 200
_NBLK = (N_NODES + N_EDGES) // _BLK
_NODE_BLKS = N_NODES // _BLK
_ITERS = -(-_NBLK // NW)
_NT_SLICE = NODE_VOCAB_PAD // NS
_ET_SLICE = EDGE_VOCAB_PAD // NS


@functools.partial(
    pl.kernel,
    out_type=jax.ShapeDtypeStruct((N_NODES + N_EDGES, DIM), jnp.float32),
    mesh=_MESH,
    scratch_types=[
        pltpu.VMEM((_BLK,), jnp.int32),
        pltpu.VMEM((_BLK,), jnp.int32),
        pltpu.VMEM((_BLK, DIM), jnp.float32),
        pltpu.VMEM((_BLK, DIM), jnp.float32),
        pltpu.SemaphoreType.DMA,
        pltpu.SemaphoreType.DMA,
        pltpu.SemaphoreType.DMA,
        pltpu.SemaphoreType.DMA,
    ],
)
def _gather(x_h, e_h, nt_h, et_h, out_h, idx0_v, idx1_v, rows0_v,
            rows1_v, g0_sem, g1_sem, w0_sem, w1_sem):
    wid = lax.axis_index("s") * NC + lax.axis_index("c")
    idx_v = (idx0_v, idx1_v)
    rows_v = (rows0_v, rows1_v)
    gsem = (g0_sem, g1_sem)
    wsem = (w0_sem, w1_sem)

    def start_block(i):
        buf = i & 1
        b = wid + NW * i

        @pl.when(b < _NBLK)
        def _():
            if i >= 2:
                pltpu.make_async_copy(
                    rows_v[buf], out_h.at[pl.ds(0, _BLK)], wsem[buf]).wait()

            @pl.when(b < _NODE_BLKS)
            def _():
                pltpu.sync_copy(x_h.at[pl.ds(b * _BLK, _BLK)],
                                idx_v[buf])
                pltpu.async_copy(nt_h.at[idx_v[buf]], rows_v[buf],
                                 gsem[buf])

            @pl.when(b >= _NODE_BLKS)
            def _():
                pltpu.sync_copy(e_h.at[pl.ds(b * _BLK - N_NODES, _BLK)],
                                idx_v[buf])
                pltpu.async_copy(et_h.at[idx_v[buf]], rows_v[buf],
                                 gsem[buf])

    def finish_block(i):
        buf = i & 1
        b = wid + NW * i

        @pl.when(b < _NBLK)
        def _():
            pltpu.make_async_copy(
                nt_h.at[pl.ds(0, _BLK)], rows_v[buf], gsem[buf]).wait()
            pltpu.async_copy(rows_v[buf], out_h.at[pl.ds(b * _BLK, _BLK)],
                             wsem[buf])

    start_block(0)
    for i in range(1, _ITERS):
        start_block(i)
        finish_block(i - 1)
    finish_block(_ITERS - 1)
    for i in (_ITERS - 2, _ITERS - 1):
        buf = i & 1
        b = wid + NW * i

        @pl.when(b < _NBLK)
        def _():
            pltpu.make_async_copy(
                rows_v[buf], out_h.at[pl.ds(0, _BLK)], wsem[buf]).wait()


def kernel(x, edge_attr, node_table, edge_table):
    cn, ce = _hist(x, edge_attr)
    ntn, etn = _stats(cn, ce, node_table, edge_table)
    return _gather(x, edge_attr, ntn, etn)


# speedup vs baseline: 1.2466x; 1.0007x over previous
import functools

import jax
import jax.numpy as jnp
from jax import lax
from jax.experimental import pallas as pl
from jax.experimental.pallas import tpu as pltpu
from jax.experimental.pallas import tpu_sc as plsc

N_NODES = 10000
N_EDGES = 160000
DIM = 256
NODE_VOCAB = 5000
EDGE_VOCAB = 500
NODE_VOCAB_PAD = 5120
EDGE_VOCAB_PAD = 512
EPS = 1e-5

NC = 2
NS = 16
NW = NC * NS
L = 16

_MESH = plsc.VectorSubcoreMesh(core_axis_name="c", subcore_axis_name="s")

_NODE_CHUNK = 320
_NODE_TAIL = N_NODES - 31 * _NODE_CHUNK
_EDGE_CHUNK = N_EDGES // NW
_EDGE_FULL = _EDGE_CHUNK // L
_EDGE_REM = _EDGE_CHUNK - _EDGE_FULL * L


@functools.partial(
    pl.kernel,
    out_type=(
        jax.ShapeDtypeStruct((NW, NODE_VOCAB_PAD), jnp.float32),
        jax.ShapeDtypeStruct((NW, EDGE_VOCAB_PAD), jnp.float32),
    ),
    mesh=_MESH,
    scratch_types=[
        pltpu.VMEM((_NODE_CHUNK,), jnp.int32),
        pltpu.VMEM((_EDGE_CHUNK + 16,), jnp.int32),
        pltpu.VMEM((NS * NODE_VOCAB_PAD,), jnp.float32),
        pltpu.VMEM((NS * EDGE_VOCAB_PAD,), jnp.float32),
        pltpu.VMEM((NODE_VOCAB_PAD,), jnp.float32),
        pltpu.VMEM((EDGE_VOCAB_PAD,), jnp.float32),
    ],
    compiler_params=pltpu.CompilerParams(needs_layout_passes=False),
)
def _hist(x_h, e_h, cn_out, ce_out, idxn_v, idxe_v, cntn_v, cnte_v, redn_v,
          rede_v):
    wid = lax.axis_index("s") * NC + lax.axis_index("c")
    lanes = jnp.arange(L, dtype=jnp.int32)
    ones = jnp.ones((L,), jnp.float32)
    zeros = jnp.zeros((L,), jnp.float32)
    lane_off_n = lanes * NODE_VOCAB_PAD
    lane_off_e = lanes * EDGE_VOCAB_PAD

    def zn(c, carry):
        cntn_v[pl.ds(c * L, L)] = zeros
        return carry
    lax.fori_loop(0, NS * NODE_VOCAB_PAD // L, zn, 0, unroll=8)
    def ze(c, carry):
        cnte_v[pl.ds(c * L, L)] = zeros
        return carry
    lax.fori_loop(0, NS * EDGE_VOCAB_PAD // L, ze, 0, unroll=8)

    @pl.when(wid < 31)
    def _():
        pltpu.sync_copy(x_h.at[pl.ds(wid * _NODE_CHUNK, _NODE_CHUNK)], idxn_v)
        def sn(i, carry):
            iv = idxn_v[pl.ds(i * L, L)]
            plsc.addupdate_scatter(cntn_v, [lane_off_n + iv], ones)
            return carry
        lax.fori_loop(0, _NODE_CHUNK // L, sn, 0)

    @pl.when(wid == 31)
    def _():
        pltpu.sync_copy(x_h.at[pl.ds(31 * _NODE_CHUNK, _NODE_TAIL)],
                        idxn_v.at[pl.ds(0, _NODE_TAIL)])
        def sn(i, carry):
            iv = idxn_v[pl.ds(i * L, L)]
            plsc.addupdate_scatter(cntn_v, [lane_off_n + iv], ones)
            return carry
        lax.fori_loop(0, _NODE_TAIL // L, sn, 0)

    pltpu.sync_copy(e_h.at[pl.ds(wid * _EDGE_CHUNK, _EDGE_CHUNK)],
                    idxe_v.at[pl.ds(0, _EDGE_CHUNK)])
    def se(i, carry):
        iv = idxe_v[pl.ds(i * L, L)]
        plsc.addupdate_scatter(cnte_v, [lane_off_e + iv], ones)
        return carry
    lax.fori_loop(0, _EDGE_FULL, se, 0, unroll=8)
    iv = idxe_v[pl.ds(_EDGE_FULL * L, L)]
    plsc.addupdate_scatter(cnte_v, [lane_off_e + iv], ones,
                           mask=lanes < _EDGE_REM)

    def rn(c, carry):
        s = pl.ds(c * L, L)
        acc = cntn_v[s]
        for r in range(1, NS):
            acc = acc + cntn_v[pl.ds(r * NODE_VOCAB_PAD + c * L, L)]
        redn_v[s] = acc
        return carry
    lax.fori_loop(0, NODE_VOCAB_PAD // L, rn, 0, unroll=4)
    for c in range(EDGE_VOCAB_PAD // L):
        s = pl.ds(c * L, L)
        acc = cnte_v[s]
        for r in range(1, NS):
            acc = acc + cnte_v[pl.ds(r * EDGE_VOCAB_PAD + c * L, L)]
        rede_v[s] = acc

    pltpu.sync_copy(redn_v, cn_out.at[wid])
    pltpu.sync_copy(rede_v, ce_out.at[wid])


def _stats_body(cn_ref, ce_ref, nt_ref, et_ref, ntn_ref, etn_ref):
    for c_ref, t_ref, o_ref, n in (
        (cn_ref, nt_ref, ntn_ref, float(N_NODES)),
        (ce_ref, et_ref, etn_ref, float(N_EDGES)),
    ):
        counts = jnp.sum(c_ref[...], axis=0, keepdims=True)[:, :t_ref.shape[0]]
        t = t_ref[...]
        mu = lax.dot(counts, t, precision=lax.Precision.HIGHEST) / n
        m2 = lax.dot(counts, t * t, precision=lax.Precision.HIGHEST) / n
        scale = lax.rsqrt(m2 - mu * mu + EPS)
        o_ref[...] = (t - mu) * scale


def _stats(cn, ce, nt, et):
    return pl.pallas_call(
        _stats_body,
        out_shape=(
            jax.ShapeDtypeStruct((NODE_VOCAB, DIM), jnp.float32),
            jax.ShapeDtypeStruct((EDGE_VOCAB, DIM), jnp.float32),
        ),
    )(cn, ce, nt, et)


_BLK = 200
_NBLK = (N_NODES + N_EDGES) // _BLK
_NODE_BLKS = N_NODES // _BLK
_ITERS = 27
_NT_SLICE = NODE_VOCAB_PAD // NS
_ET_SLICE = EDGE_VOCAB_PAD // NS


@functools.partial(
    pl.kernel,
    out_type=jax.ShapeDtypeStruct((N_NODES + N_EDGES, DIM), jnp.float32),
    mesh=_MESH,
    scratch_types=[
        pltpu.VMEM((_BLK,), jnp.int32),
        pltpu.VMEM((_BLK,), jnp.int32),
        pltpu.VMEM((_BLK, DIM), jnp.float32),
        pltpu.VMEM((_BLK, DIM), jnp.float32),
        pltpu.SemaphoreType.DMA,
        pltpu.SemaphoreType.DMA,
        pltpu.SemaphoreType.DMA,
        pltpu.SemaphoreType.DMA,
    ],
)
def _gather(x_h, e_h, nt_h, et_h, out_h, idx0_v, idx1_v, rows0_v,
            rows1_v, g0_sem, g1_sem, w0_sem, w1_sem):
    wid = lax.axis_index("s") * NC + lax.axis_index("c")
    idx_v = (idx0_v, idx1_v)
    rows_v = (rows0_v, rows1_v)
    gsem = (g0_sem, g1_sem)
    wsem = (w0_sem, w1_sem)

    def block_of(i):
        if i < 2:
            return wid + NW * i, (wid + NW * i) < _NODE_BLKS
        return _NODE_BLKS + wid * 25 + (i - 2), True

    def start_block(i):
        buf = i & 1
        b, valid = block_of(i)

        if i >= 2:
            _, vprev = block_of(i - 2)

            @pl.when(vprev)
            def _():
                pltpu.make_async_copy(
                    rows_v[buf], out_h.at[pl.ds(0, _BLK)], wsem[buf]).wait()

        @pl.when(valid)
        def _():
            if i < 2:
                pltpu.sync_copy(x_h.at[pl.ds(b * _BLK, _BLK)], idx_v[buf])
                pltpu.async_copy(nt_h.at[idx_v[buf]], rows_v[buf], gsem[buf])
            else:
                pltpu.sync_copy(e_h.at[pl.ds(b * _BLK - N_NODES, _BLK)],
                                idx_v[buf])
                pltpu.async_copy(et_h.at[idx_v[buf]], rows_v[buf], gsem[buf])

    def finish_block(i):
        buf = i & 1
        b, valid = block_of(i)

        @pl.when(valid)
        def _():
            pltpu.make_async_copy(
                nt_h.at[pl.ds(0, _BLK)], rows_v[buf], gsem[buf]).wait()
            pltpu.async_copy(rows_v[buf], out_h.at[pl.ds(b * _BLK, _BLK)],
                             wsem[buf])

    start_block(0)
    for i in range(1, _ITERS):
        start_block(i)
        finish_block(i - 1)
    finish_block(_ITERS - 1)
    for i in (_ITERS - 2, _ITERS - 1):
        buf = i & 1
        _, valid = block_of(i)

        @pl.when(valid)
        def _():
            pltpu.make_async_copy(
                rows_v[buf], out_h.at[pl.ds(0, _BLK)], wsem[buf]).wait()


def kernel(x, edge_attr, node_table, edge_table):
    cn, ce = _hist(x, edge_attr)
    ntn, etn = _stats(cn, ce, node_table, edge_table)
    return _gather(x, edge_attr, ntn, etn)
